# baseline (speedup 1.0000x reference)
import jax
import jax.numpy as jnp
from jax import lax
from jax.experimental import pallas as pl

_PATCH = (16, 128, 128)


def _conv3d(x, w, b, strides):
    return lax.conv_general_dilated(x, w, window_strides=strides, padding='SAME',
                                    dimension_numbers=('NDHWC', 'DHWIO', 'NDHWC')) + b


def _sigmoid_pallas(x):
    def body(x_ref, o_ref):
        o_ref[...] = jax.nn.sigmoid(x_ref[...])
    return pl.pallas_call(body, out_shape=jax.ShapeDtypeStruct(x.shape, x.dtype))(x)


def kernel(ret, machine_labels, visited, focus, W1, b1, W2, b2, W3, b3, W4, b4):
    pz, py, px = _PATCH
    z, y, x = focus[0], focus[1], focus[2]
    start = (0, z, y, x, 0)
    ml_patch = lax.dynamic_slice(machine_labels, start, (1, pz, py, px, 1))
    centre = ml_patch[0, pz // 2, py // 2, px // 2, 0]
    glimpse = (ml_patch == centre).astype(jnp.float32)
    h = jax.nn.relu(_conv3d(glimpse, W1, b1, (1, 2, 2)))
    h = jax.nn.relu(_conv3d(h, W2, b2, (2, 2, 2)))
    h = jax.nn.relu(_conv3d(h, W3, b3, (2, 2, 2)))
    logits = _conv3d(h, W4, b4, (1, 2, 2))
    otpt = _sigmoid_pallas(logits.reshape(4, 8, 8)).reshape(1, 4, 8, 8, 1)
    up = jnp.repeat(jnp.repeat(jnp.repeat(otpt, 4, axis=1), 16, axis=2), 16, axis=3)
    ret_patch = lax.dynamic_slice(ret, start, (1, pz, py, px, 1))
    vis_patch = lax.dynamic_slice(visited, start, (1, pz, py, px, 1))
    new_ret_patch = jnp.maximum(up * glimpse, ret_patch)
    new_vis_patch = vis_patch + glimpse.astype(jnp.int32)
    updated_ret = lax.dynamic_update_slice(ret, new_ret_patch, start)
    updated_vis = lax.dynamic_update_slice(visited, new_vis_patch, start)
    v0 = lax.dynamic_slice(visited, start, (1, 1, 1, 1, 1))[0, 0, 0, 0, 0]
    do = v0 <= 3
    new_ret = jnp.where(do, updated_ret, ret)
    new_visited = jnp.where(do, updated_vis, visited)
    return new_ret, new_visited


# speedup vs baseline: 1.9206x; 1.9206x over previous
import jax
import jax.numpy as jnp
from jax import lax
from jax.experimental import pallas as pl
from jax.experimental.pallas import tpu as pltpu

F32 = jnp.float32
_FULL = (1, 32, 384, 384, 1)


def _gather_body(ml_ref, vis_ref, focus_ref, gp_ref, do_ref,
                 slab, visrow, sem1, sem2):
    z, y, x = focus_ref[0], focus_ref[1], focus_ref[2]
    c1 = pltpu.make_async_copy(
        ml_ref.at[pl.ds(z, 16), pl.ds(0, 144), :], slab, sem1)
    c1.start()
    c2 = pltpu.make_async_copy(
        vis_ref.at[pl.ds(z, 1), pl.ds(0, 144), :], visrow, sem2)
    c2.start()
    c1.wait()
    c2.wait()
    slab_f = slab[...].astype(F32)
    s8 = slab_f[8:9, :, :].reshape(144, 384)
    i0 = lax.broadcasted_iota(jnp.int32, (144, 384), 0)
    i1 = lax.broadcasted_iota(jnp.int32, (144, 384), 1)
    cmask = ((i0 == y + 64) & (i1 == x + 64)).astype(F32)
    centre = jnp.sum(s8 * cmask)
    eq = (slab_f == centre).astype(F32).reshape(2304, 384)
    rowk = lax.broadcasted_iota(jnp.int32, (384, 128), 0)
    colj = lax.broadcasted_iota(jnp.int32, (384, 128), 1)
    sx = (rowk == colj + x).astype(F32)
    exq = jnp.dot(eq, sx, preferred_element_type=F32).reshape(16, 144, 128)
    rj = lax.broadcasted_iota(jnp.int32, (128, 144), 0)
    ck = lax.broadcasted_iota(jnp.int32, (128, 144), 1)
    sy = (ck == y + rj).astype(F32)
    parts = []
    for zz in range(16):
        parts.append(jnp.dot(sy, exq[zz], preferred_element_type=F32))
    gp_ref[...] = jnp.concatenate(parts, axis=0)
    vr = visrow[...].astype(F32).reshape(144, 384)
    vmask = ((i0 == y) & (i1 == x)).astype(F32)
    v0 = jnp.sum(vr * vmask, axis=(0, 1), keepdims=True).reshape(1, 1)
    do_ref[...] = jnp.where(v0 <= 3.5, 1.0, 0.0)


def _gather(ml3, vis3, focus):
    return pl.pallas_call(
        _gather_body,
        in_specs=[pl.BlockSpec(memory_space=pl.ANY),
                  pl.BlockSpec(memory_space=pl.ANY),
                  pl.BlockSpec(memory_space=pltpu.SMEM)],
        out_specs=[pl.BlockSpec(memory_space=pltpu.VMEM),
                   pl.BlockSpec(memory_space=pltpu.VMEM)],
        out_shape=[jax.ShapeDtypeStruct((2048, 128), F32),
                   jax.ShapeDtypeStruct((1, 1), F32)],
        scratch_shapes=[pltpu.VMEM((16, 144, 384), jnp.int32),
                        pltpu.VMEM((1, 144, 384), jnp.int32),
                        pltpu.SemaphoreType.DMA, pltpu.SemaphoreType.DMA],
    )(ml3, vis3, focus)


def _l1_body(x_ref, w_ref, b_ref, o_ref):
    acc = jnp.zeros((1024, 2048), F32) + b_ref[...]
    for t in range(9):
        acc = acc + jnp.dot(x_ref[t], w_ref[t], preferred_element_type=F32)
    o_ref[...] = jnp.maximum(acc, 0.0)


def _l2_body(x_ref, w_ref, b_ref, o_ref):
    acc = jnp.zeros((8192, 64), F32) + b_ref[...]
    t = 0
    for jz in range(2):
        for jy in range(2):
            for jx in range(2):
                src = x_ref[jx, jz:jz + 8, jy:jy + 32, :, :]
                acc = acc + jnp.dot(src.reshape(8192, 256), w_ref[t],
                                    preferred_element_type=F32)
                t += 1
    o_ref[...] = jnp.maximum(acc, 0.0)


def _l3_body(x_ref, w_ref, b_ref, o_ref):
    acc = jnp.zeros((1024, 128), F32) + b_ref[...]
    t = 0
    for jz in range(2):
        for jy in range(2):
            for jx in range(2):
                src = x_ref[jx, jz:jz + 4, jy:jy + 16, :, :]
                acc = acc + jnp.dot(src.reshape(1024, 512), w_ref[t],
                                    preferred_element_type=F32)
                t += 1
    o_ref[...] = jnp.maximum(acc, 0.0)


def _conv_call(body, x, w, b, out_rows, out_ch):
    return pl.pallas_call(
        body,
        out_shape=jax.ShapeDtypeStruct((out_rows, out_ch), F32),
    )(x, w, b)


def _final_body(ret_ref, vis_ref, x4_ref, w4_ref, b4_ref, gp_ref, do_ref,
                focus_ref, oret_ref, ovis_ref, rslab, vslab,
                sem1, sem2, sem3, sem4):
    z, y, x = focus_ref[0], focus_ref[1], focus_ref[2]
    c1 = pltpu.make_async_copy(
        ret_ref.at[pl.ds(z, 16), pl.ds(0, 144), :], rslab, sem1)
    c1.start()
    c2 = pltpu.make_async_copy(
        vis_ref.at[pl.ds(z, 16), pl.ds(0, 144), :], vslab, sem2)
    c2.start()
    acc = jnp.zeros((256, 1), F32) + b4_ref[...]
    t = 0
    for dz in range(3):
        for jy in range(2):
            for jx in range(2):
                src = x4_ref[jx, dz:dz + 4, jy:jy + 8, :, :]
                acc = acc + jnp.dot(src.reshape(256, 512), w4_ref[t],
                                    preferred_element_type=F32)
                t += 1
    p3 = (1.0 / (1.0 + jnp.exp(-acc))).reshape(4, 8, 8)
    yy = lax.broadcasted_iota(jnp.int32, (128, 8), 0)
    kk = lax.broadcasted_iota(jnp.int32, (128, 8), 1)
    ey = ((yy // 16) == kk).astype(F32)
    kk2 = lax.broadcasted_iota(jnp.int32, (8, 128), 0)
    xx = lax.broadcasted_iota(jnp.int32, (8, 128), 1)
    ex = (kk2 == (xx // 16)).astype(F32)
    ups = []
    for zc in range(4):
        a = jnp.dot(ey, p3[zc], preferred_element_type=F32)
        ups.append(jnp.dot(a, ex, preferred_element_type=F32)
                   .reshape(1, 128, 128))
    up4 = jnp.concatenate(ups, axis=0)
    up = jnp.broadcast_to(up4[:, None], (4, 4, 128, 128)).reshape(16, 128, 128)
    gp = gp_ref[...].reshape(16, 128, 128)
    dof = do_ref[...].reshape(1, 1, 1)
    contrib = up * gp * dof
    gpd = gp * dof
    rk = lax.broadcasted_iota(jnp.int32, (144, 128), 0)
    cj = lax.broadcasted_iota(jnp.int32, (144, 128), 1)
    syt = (rk == y + cj).astype(F32)
    jj = lax.broadcasted_iota(jnp.int32, (128, 384), 0)
    kk3 = lax.broadcasted_iota(jnp.int32, (128, 384), 1)
    sxt = (kk3 == jj + x).astype(F32)
    rparts, vparts = [], []
    for zz in range(16):
        a = jnp.dot(syt, contrib[zz], preferred_element_type=F32)
        rparts.append(jnp.dot(a, sxt, preferred_element_type=F32)
                      .reshape(1, 144, 384))
        b = jnp.dot(syt, gpd[zz], preferred_element_type=F32)
        vparts.append(jnp.dot(b, sxt, preferred_element_type=F32)
                      .reshape(1, 144, 384))
    rc = jnp.concatenate(rparts, axis=0)
    vc = jnp.concatenate(vparts, axis=0)
    c1.wait()
    c2.wait()
    rslab[...] = jnp.maximum(rslab[...], rc)
    vslab[...] = vslab[...] + vc.astype(jnp.int32)
    co1 = pltpu.make_async_copy(
        rslab, oret_ref.at[pl.ds(z, 16), pl.ds(0, 144), :], sem3)
    co1.start()
    co2 = pltpu.make_async_copy(
        vslab, ovis_ref.at[pl.ds(z, 16), pl.ds(0, 144), :], sem4)
    co2.start()
    co1.wait()
    co2.wait()


def _final(ret3, vis3, x4, w4e, b4e, gp2, doflag, focus):
    return pl.pallas_call(
        _final_body,
        in_specs=[pl.BlockSpec(memory_space=pl.ANY),
                  pl.BlockSpec(memory_space=pl.ANY),
                  pl.BlockSpec(memory_space=pltpu.VMEM),
                  pl.BlockSpec(memory_space=pltpu.VMEM),
                  pl.BlockSpec(memory_space=pltpu.VMEM),
                  pl.BlockSpec(memory_space=pltpu.VMEM),
                  pl.BlockSpec(memory_space=pltpu.VMEM),
                  pl.BlockSpec(memory_space=pltpu.SMEM)],
        out_specs=[pl.BlockSpec(memory_space=pl.ANY),
                   pl.BlockSpec(memory_space=pl.ANY)],
        out_shape=[jax.ShapeDtypeStruct((32, 384, 384), F32),
                   jax.ShapeDtypeStruct((32, 384, 384), jnp.int32)],
        input_output_aliases={0: 0, 1: 1},
        scratch_shapes=[pltpu.VMEM((16, 144, 384), F32),
                        pltpu.VMEM((16, 144, 384), jnp.int32),
                        pltpu.SemaphoreType.DMA, pltpu.SemaphoreType.DMA,
                        pltpu.SemaphoreType.DMA, pltpu.SemaphoreType.DMA],
    )(ret3, vis3, x4, w4e, b4e, gp2, doflag, focus)


def _s2d(a, f):
    d, h, w, c = a.shape
    fz, fy, fx = f
    a = a.reshape(d // fz, fz, h // fy, fy, w // fx, fx, c)
    a = a.transpose(0, 2, 4, 1, 3, 5, 6)
    return a.reshape(d // fz, h // fy, w // fx, fz * fy * fx * c)


def _xpair(a, wout):
    return jnp.stack([a[:, :, 0:wout, :], a[:, :, 1:wout + 1, :]], axis=0)


def kernel(ret, machine_labels, visited, focus, W1, b1, W2, b2, W3, b3, W4, b4):
    ml3 = machine_labels.reshape(32, 384, 384)
    ret3 = ret.reshape(32, 384, 384)
    vis3 = visited.reshape(32, 384, 384)
    focus = focus.astype(jnp.int32)

    gp2, doflag = _gather(ml3, vis3, focus)

    iox = jnp.arange(64)
    ixx = jnp.arange(128)
    idx = jnp.arange(3)
    xsel = (ixx[None, :, None] == 2 * iox[None, None, :]
            + idx[:, None, None]).astype(F32)
    w1e = jnp.einsum('axo,zyac->zyxoc', xsel,
                     W1[:, :, :, 0, :]).reshape(9, 128, 2048)
    b1e = jnp.tile(b1, 64).reshape(1, 2048)
    w2 = jnp.pad(W2, ((0, 1), (0, 1), (0, 1), (0, 0), (0, 0)))
    w2e = (w2.reshape(2, 2, 2, 2, 2, 2, 32, 64)
           .transpose(0, 2, 4, 1, 3, 5, 6, 7).reshape(8, 256, 64))
    w3 = jnp.pad(W3, ((0, 1), (0, 1), (0, 1), (0, 0), (0, 0)))
    w3e = (w3.reshape(2, 2, 2, 2, 2, 2, 64, 128)
           .transpose(0, 2, 4, 1, 3, 5, 6, 7).reshape(8, 512, 128))
    w4 = jnp.pad(W4[:, :, :, :, 0], ((0, 0), (0, 1), (0, 1), (0, 0)))
    w4e = (w4.reshape(3, 2, 2, 2, 2, 128).transpose(0, 1, 3, 2, 4, 5)
           .reshape(12, 512, 1))

    gpz = jnp.pad(gp2.reshape(16, 128, 128), ((1, 1), (0, 2), (0, 0)))
    a1 = jnp.stack([gpz[dz:dz + 16, dy:dy + 128:2, :]
                    for dz in range(3) for dy in range(3)], axis=0)
    out1 = _conv_call(_l1_body, a1.reshape(9, 1024, 128), w1e, b1e,
                      1024, 2048)

    x2 = _s2d(jnp.pad(out1.reshape(16, 64, 64, 32),
                      ((0, 2), (0, 2), (0, 2), (0, 0))), (2, 2, 2))
    out2 = _conv_call(_l2_body, _xpair(x2, 32), w2e, b2.reshape(1, 64),
                      8192, 64)

    x3 = _s2d(jnp.pad(out2.reshape(8, 32, 32, 64),
                      ((0, 2), (0, 2), (0, 2), (0, 0))), (2, 2, 2))
    out3 = _conv_call(_l3_body, _xpair(x3, 16), w3e, b3.reshape(1, 128),
                      1024, 128)

    x4 = _s2d(jnp.pad(out3.reshape(4, 16, 16, 128),
                      ((1, 1), (0, 2), (0, 2), (0, 0))), (1, 2, 2))

    ret_o, vis_o = _final(ret3, vis3, _xpair(x4, 8), w4e, b4.reshape(1, 1),
                          gp2, doflag, focus)
    return ret_o.reshape(_FULL), vis_o.reshape(_FULL)
